# baseline (speedup 1.0000x reference)
import jax
import jax.numpy as jnp
from jax.experimental import pallas as pl
from jax.experimental.pallas import tpu as pltpu

N_BLOCK = 1024


def _c2_kernel(centers_ref, c2_ref):
    c = centers_ref[...]
    c2_ref[...] = jnp.sum(c * c, axis=1)[None, :]


def _cluster_kernel(embs_ref, centers_ref, c2_ref, idx_ref, ids_ref, mind_ref):
    c = centers_ref[...]
    e = embs_ref[...]
    ndot2 = jax.lax.dot_general(
        e * -2.0, c, (((1,), (1,)), ((), ())),
        preferred_element_type=jnp.float32)
    e2 = jnp.sum(e * e, axis=1, keepdims=True)
    d2 = (e2 + ndot2) + c2_ref[...]
    idx = idx_ref[...]

    w = d2.shape[1] // 2
    take_r = d2[:, w:] < d2[:, :w]
    vals = jnp.minimum(d2[:, :w], d2[:, w:])
    iis = jnp.where(take_r, idx[:, w:], idx[:, :w])
    while w > 128:
        w //= 2
        take_r = vals[:, w:] < vals[:, :w]
        vals = jnp.minimum(vals[:, :w], vals[:, w:])
        iis = jnp.where(take_r, iis[:, w:], iis[:, :w])

    row_min = jnp.min(vals, axis=1, keepdims=True)
    ids_f = jnp.min(jnp.where(vals == row_min, iis, jnp.inf), axis=1)
    ids_ref[...] = ids_f.astype(jnp.int32)[None, None, :]
    mind = jnp.sqrt(jnp.maximum(row_min, 0.0) + 1e-12)
    mind_ref[...] = mind[:, 0][None, None, :]


def _loss_kernel(mind_ref, loss_ref):
    loss_ref[0, 0] = jnp.sum(mind_ref[...])


def kernel(embs, centers):
    n, d = embs.shape
    k = centers.shape[0]
    grid = n // N_BLOCK

    c2 = pl.pallas_call(
        _c2_kernel,
        in_specs=[pl.BlockSpec((k, d), lambda: (0, 0))],
        out_specs=pl.BlockSpec((1, k), lambda: (0, 0)),
        out_shape=jax.ShapeDtypeStruct((1, k), jnp.float32),
    )(centers)

    ids, mind = pl.pallas_call(
        _cluster_kernel,
        grid=(grid,),
        in_specs=[
            pl.BlockSpec((N_BLOCK, d), lambda i: (i, 0)),
            pl.BlockSpec((k, d), lambda i: (0, 0)),
            pl.BlockSpec((1, k), lambda i: (0, 0)),
            pl.BlockSpec((1, k), lambda i: (0, 0)),
        ],
        out_specs=[
            pl.BlockSpec((1, 1, N_BLOCK), lambda i: (i, 0, 0)),
            pl.BlockSpec((1, 1, N_BLOCK), lambda i: (i, 0, 0)),
        ],
        out_shape=[
            jax.ShapeDtypeStruct((grid, 1, N_BLOCK), jnp.int32),
            jax.ShapeDtypeStruct((grid, 1, N_BLOCK), jnp.float32),
        ],
        compiler_params=pltpu.CompilerParams(
            dimension_semantics=("parallel",)),
    )(embs, centers, c2, jnp.arange(k, dtype=jnp.float32)[None, :])

    loss = pl.pallas_call(
        _loss_kernel,
        in_specs=[pl.BlockSpec((grid, 1, N_BLOCK), lambda: (0, 0, 0))],
        out_specs=pl.BlockSpec((1, 1), lambda: (0, 0),
                               memory_space=pltpu.SMEM),
        out_shape=jax.ShapeDtypeStruct((1, 1), jnp.float32),
    )(mind)

    return (centers, ids.reshape(n), loss[0, 0])


# speedup vs baseline: 1.1949x; 1.1949x over previous
import jax
import jax.numpy as jnp
from jax.experimental import pallas as pl
from jax.experimental.pallas import tpu as pltpu

N_BLOCK = 1024


def _c2_kernel(centers_ref, c2_ref):
    c = centers_ref[...]
    c2_ref[...] = jnp.sum(c * c, axis=1)[None, :]


def _cluster_kernel(embs_ref, centers_ref, c2_ref, idx_ref, ids_ref, loss_ref):
    c = centers_ref[...]
    e = embs_ref[...]
    ndot2 = jax.lax.dot_general(
        e * -2.0, c, (((1,), (1,)), ((), ())),
        preferred_element_type=jnp.float32)
    e2 = jnp.sum(e * e, axis=1, keepdims=True)
    d2 = (e2 + ndot2) + c2_ref[...]
    row_min = jnp.min(d2, axis=1, keepdims=True)
    idx = idx_ref[...]
    ids_f = jnp.min(jnp.where(d2 == row_min, idx, jnp.inf), axis=1)
    ids_ref[...] = ids_f.astype(jnp.int32)[None, None, :]
    mind = jnp.sqrt(jnp.maximum(row_min, 0.0) + 1e-12)
    partial = jnp.sum(mind)

    @pl.when(pl.program_id(0) == 0)
    def _init():
        loss_ref[0, 0] = partial

    @pl.when(pl.program_id(0) != 0)
    def _acc():
        loss_ref[0, 0] += partial


def kernel(embs, centers):
    n, d = embs.shape
    k = centers.shape[0]
    grid = n // N_BLOCK

    c2 = pl.pallas_call(
        _c2_kernel,
        in_specs=[pl.BlockSpec((k, d), lambda: (0, 0))],
        out_specs=pl.BlockSpec((1, k), lambda: (0, 0)),
        out_shape=jax.ShapeDtypeStruct((1, k), jnp.float32),
    )(centers)

    ids, loss = pl.pallas_call(
        _cluster_kernel,
        grid=(grid,),
        in_specs=[
            pl.BlockSpec((N_BLOCK, d), lambda i: (i, 0)),
            pl.BlockSpec((k, d), lambda i: (0, 0)),
            pl.BlockSpec((1, k), lambda i: (0, 0)),
            pl.BlockSpec((1, k), lambda i: (0, 0)),
        ],
        out_specs=[
            pl.BlockSpec((1, 1, N_BLOCK), lambda i: (i, 0, 0)),
            pl.BlockSpec((1, 1), lambda i: (0, 0), memory_space=pltpu.SMEM),
        ],
        out_shape=[
            jax.ShapeDtypeStruct((grid, 1, N_BLOCK), jnp.int32),
            jax.ShapeDtypeStruct((1, 1), jnp.float32),
        ],
    )(embs, centers, c2, jnp.arange(k, dtype=jnp.float32)[None, :])

    return (centers, ids.reshape(n), loss[0, 0])
